# baseline (speedup 1.0000x reference)
import functools
import jax
import jax.numpy as jnp
from jax import lax
from jax.experimental import pallas as pl
from jax.experimental.pallas import tpu as pltpu
from jax.experimental.pallas import tpu_sc as plsc

REGIONS = 16
NUM_CABIN = 8
SP_RATIO = 4
L = 16
B, F, N = 8, 128, 8192
P = N // SP_RATIO
NV = N // L
CH = REGIONS + 3
IOFF = 8320


def _sorter_body(x_ref, w_ref, b_ref, va_ref, ida_ref):
    x = x_ref[0]
    w = w_ref[...]
    va = jnp.dot(w, x, preferred_element_type=jnp.float32) + b_ref[...][:, None]
    va_ref[0] = va
    ida_ref[0, 0] = jnp.argmax(va, axis=0).astype(jnp.int32)


def _sorter(x, w_sorter, b_sorter):
    R = REGIONS
    return pl.pallas_call(
        _sorter_body,
        grid=(B,),
        in_specs=[
            pl.BlockSpec((1, F, N), lambda b: (b, 0, 0)),
            pl.BlockSpec((R, F), lambda b: (0, 0)),
            pl.BlockSpec((R,), lambda b: (0,)),
        ],
        out_specs=[
            pl.BlockSpec((1, R, N), lambda b: (b, 0, 0)),
            pl.BlockSpec((1, 1, N), lambda b: (b, 0, 0)),
        ],
        out_shape=[
            jax.ShapeDtypeStruct((B, REGIONS, N), jnp.float32),
            jax.ShapeDtypeStruct((B, 1, N), jnp.int32),
        ],
    )(x, w_sorter, b_sorter)


def _i32(x):
    return lax.bitcast_convert_type(x, jnp.int32)


def _f32(x):
    return lax.bitcast_convert_type(x, jnp.float32)


def _make_sc_body():
    def body(x_hbm, va_hbm, cube_hbm, spidx_hbm, cab_hbm,
             fb0, fb1, ob0, ob1, cnt, msc, ib, cb0, cb1,
             si0, si1, so0, so1, idx_sp):
        _sc_impl(x_hbm, va_hbm, cube_hbm, spidx_hbm, cab_hbm,
                 fb0, fb1, ob0, ob1, cnt, msc, ib, cb0, cb1,
                 si0, si1, so0, so1, idx_sp)
    return body


def _sc_impl(x_hbm, va_hbm, cube_hbm, spidx_hbm, cab_hbm,
             fb0, fb1, ob0, ob1, cnt, msc, ib, cb0, cb1,
             si0, si1, so0, so1, idx_sp):
    cid = lax.axis_index("c")
    sid = lax.axis_index("s")
    lanes = lax.iota(jnp.int32, L)
    ones = jnp.ones((L,), jnp.int32)
    lane0 = lanes == 0

    def key_at(v):
        u = _i32(fb0[pl.ds(v * L, L)])
        s = lax.shift_right_arithmetic(u, 31)
        return u ^ (jnp.bitwise_not(s) & jnp.int32(0x7FFFFFFF))

    def sort_row(row_j, carry):
        row_local = sid * 4 + row_j
        b_local = row_local // REGIONS
        r = row_local % REGIONS
        b = cid * 4 + b_local
        pltpu.sync_copy(va_hbm.at[b, r], fb0)

        def z256(v, c):
            cnt[pl.ds(v * L, L)] = jnp.zeros((L,), jnp.int32)
            return c
        lax.fori_loop(0, 256, z256, 0)

        def h256(v, c):
            d0 = lax.shift_right_logical(key_at(2 * v), 24)
            plsc.addupdate_scatter(cnt, [lanes * 256 + d0], ones)
            d1 = lax.shift_right_logical(key_at(2 * v + 1), 24)
            plsc.addupdate_scatter(cnt, [lanes * 256 + d1], ones)
            return c
        lax.fori_loop(0, NV // 2, h256, 0)

        tot_prev = jnp.int32(0)
        nge = jnp.int32(0)
        for chunk in range(16):
            tv = jnp.zeros((L,), jnp.int32)
            for l in range(L):
                tv = tv + cnt[pl.ds(l * 256 + chunk * L, L)]
            incl = plsc.cumsum(tv) + tot_prev
            msc[pl.ds(chunk * L, L)] = incl
            nge = nge + jnp.max(plsc.all_reduce_population_count(incl >= P))
            tot_prev = jnp.max(incl)
        bc = jnp.int32(256) - nge
        nc = jnp.max(plsc.load_gather(msc, [jnp.full((L,), bc, jnp.int32)]))

        def cp_one(v, off):
            key = key_at(v)
            pay = jnp.full((L,), v * L, jnp.int32) + lanes
            m = lax.shift_right_logical(key, 24) <= bc
            msel = jnp.where(m, ones, jnp.zeros((L,), jnp.int32))
            excl = plsc.cumsum(msel) - msel
            addr = jnp.full((L,), off, jnp.int32) + excl
            plsc.store_scatter(ob1, [addr], _f32(key), mask=m)
            plsc.store_scatter(ob1, [addr + IOFF], _f32(pay), mask=m)
            return off + jnp.max(plsc.all_reduce_population_count(m))

        def cp(v, off):
            return cp_one(2 * v + 1, cp_one(2 * v, off))
        nc2 = lax.fori_loop(0, NV // 2, cp, jnp.int32(0))
        for pad in range(2):
            padd = jnp.full((L,), nc2 + pad * L, jnp.int32) + lanes
            plsc.store_scatter(ob1, [padd], _f32(jnp.full((L,), -1, jnp.int32)))
            plsc.store_scatter(ob1, [padd + IOFF], jnp.zeros((L,), jnp.float32))
        nvh = lax.shift_right_logical(nc + jnp.int32(31), 5)
        nv = nvh * 2

        def radix_pass(src, dst, sh):
            shv = jnp.full((L,), sh, jnp.int32)
            lnv = lanes * nv
            for l in range(L):
                cnt[pl.ds(l * L, L)] = jnp.zeros((L,), jnp.int32)

            def h_one(pos):
                key = _i32(plsc.load_gather(src, [pos]))
                d = jnp.bitwise_and(lax.shift_right_logical(key, shv), 15)
                plsc.addupdate_scatter(cnt, [lanes * L + d], ones)

            def h(v, c):
                pos = jnp.full((L,), 2 * v, jnp.int32) + lnv
                h_one(pos)
                h_one(pos + 1)
                return c
            lax.fori_loop(0, nvh, h, 0)

            cs = []
            tv = jnp.zeros((L,), jnp.int32)
            for l in range(L):
                cv = cnt[pl.ds(l * L, L)]
                cs.append(cv)
                tv = tv + cv
            run = plsc.cumsum(tv) - tv
            for l in range(L):
                cnt[pl.ds(l * L, L)] = run
                run = run + cs[l]

            def p_one(pos):
                keyf = plsc.load_gather(src, [pos])
                payf = plsc.load_gather(src, [pos + IOFF])
                d = jnp.bitwise_and(lax.shift_right_logical(_i32(keyf), shv), 15)
                addr = lanes * L + d
                cur = plsc.load_gather(cnt, [addr])
                plsc.store_scatter(dst, [cur], keyf)
                plsc.store_scatter(dst, [cur + IOFF], payf)
                plsc.addupdate_scatter(cnt, [addr], ones)

            def perm(v, c):
                pos = jnp.full((L,), 2 * v, jnp.int32) + lnv
                p_one(pos)
                p_one(pos + 1)
                return c
            lax.fori_loop(0, nvh, perm, 0)

        def dbl(it, c):
            radix_pass(ob1, ob0, it * 8)
            radix_pass(ob0, ob1, it * 8 + 4)
            return c
        lax.fori_loop(0, 4, dbl, 0)

        pltpu.sync_copy(ob1.at[pl.ds(IOFF, P)],
                        idx_sp.at[b_local, pl.ds(r * P, P)])
        return carry

    with jax.named_scope("sc_sort"):
        lax.fori_loop(0, 4, sort_row, 0)
    plsc.subcore_barrier()

    b_local = sid // 4
    fgrp = sid % 4
    b = cid * 4 + b_local
    pltpu.sync_copy(idx_sp.at[b_local], ib)

    def cvt(v, c):
        ob0[pl.ds(v * L, L)] = _i32(ib[pl.ds(v * L, L)]).astype(jnp.float32)
        return c
    lax.fori_loop(0, (REGIONS * P) // L, cvt, 0)
    for k in range(5):
        ch = fgrp + 4 * k

        @pl.when(ch < CH)
        def _():
            pltpu.sync_copy(ob0, spidx_hbm.at[b, ch])

    f_base = fgrp * 32

    def gather_f(ob, cb, fbuf):
        def per_rg(rg, c2):
            r = rg // NUM_CABIN
            g = rg % NUM_CABIN
            base = r * P + g * 256
            acc = jnp.full((L,), -jnp.inf, jnp.float32)
            for t in range(16):
                iv = _i32(ib[pl.ds(base + t * L, L)])
                gv = plsc.load_gather(fbuf, [iv])
                ob[pl.ds(base + t * L, L)] = gv
                acc = jnp.maximum(acc, gv)
            m = jnp.max(acc)
            plsc.store_scatter(cb, [jnp.full((L,), rg, jnp.int32)],
                               jnp.full((L,), m, jnp.float32), mask=lane0)
            return c2
        lax.fori_loop(0, REGIONS * NUM_CABIN, per_rg, 0)

    pltpu.async_copy(x_hbm.at[b, f_base], fb0, si0)

    def per_pair(i, c):
        fe = f_base + 2 * i
        fo = fe + 1
        pltpu.make_async_copy(x_hbm.at[b, fe], fb0, si0).wait()
        pltpu.async_copy(x_hbm.at[b, fo], fb1, si1)

        @pl.when(i > 0)
        def _():
            pltpu.make_async_copy(ob0, cube_hbm.at[b, fe - 2], so0).wait()
            pltpu.make_async_copy(cb0, cab_hbm.at[b, fe - 2], so0).wait()
        gather_f(ob0, cb0, fb0)
        pltpu.async_copy(ob0, cube_hbm.at[b, fe], so0)
        pltpu.async_copy(cb0, cab_hbm.at[b, fe], so0)

        pltpu.make_async_copy(x_hbm.at[b, fo], fb1, si1).wait()

        @pl.when(i < 15)
        def _():
            pltpu.async_copy(x_hbm.at[b, fe + 2], fb0, si0)

        @pl.when(i > 0)
        def _():
            pltpu.make_async_copy(ob1, cube_hbm.at[b, fo - 2], so1).wait()
            pltpu.make_async_copy(cb1, cab_hbm.at[b, fo - 2], so1).wait()
        gather_f(ob1, cb1, fb1)
        pltpu.async_copy(ob1, cube_hbm.at[b, fo], so1)
        pltpu.async_copy(cb1, cab_hbm.at[b, fo], so1)
        return c

    with jax.named_scope("sc_gather"):
        lax.fori_loop(0, 16, per_pair, 0)
    pltpu.make_async_copy(ob0, cube_hbm.at[b, f_base + 30], so0).wait()
    pltpu.make_async_copy(cb0, cab_hbm.at[b, f_base + 30], so0).wait()
    pltpu.make_async_copy(ob1, cube_hbm.at[b, f_base + 31], so1).wait()
    pltpu.make_async_copy(cb1, cab_hbm.at[b, f_base + 31], so1).wait()


def _sc_call(x, va):
    mesh = plsc.VectorSubcoreMesh(core_axis_name="c", subcore_axis_name="s")
    fn = pl.kernel(
        _make_sc_body(),
        out_type=[
            jax.ShapeDtypeStruct((B, F, REGIONS * P), jnp.float32),
            jax.ShapeDtypeStruct((B, CH, REGIONS * P), jnp.float32),
            jax.ShapeDtypeStruct((B, F, REGIONS * NUM_CABIN), jnp.float32),
        ],
        mesh=mesh,
        compiler_params=pltpu.CompilerParams(needs_layout_passes=False),
        scratch_types=[
            pltpu.VMEM((N,), jnp.float32),
            pltpu.VMEM((N,), jnp.float32),
            pltpu.VMEM((REGIONS * P,), jnp.float32),
            pltpu.VMEM((REGIONS * P,), jnp.float32),
            pltpu.VMEM((4096,), jnp.int32),
            pltpu.VMEM((256,), jnp.int32),
            pltpu.VMEM((REGIONS * P,), jnp.float32),
            pltpu.VMEM((REGIONS * NUM_CABIN,), jnp.float32),
            pltpu.VMEM((REGIONS * NUM_CABIN,), jnp.float32),
            pltpu.SemaphoreType.DMA,
            pltpu.SemaphoreType.DMA,
            pltpu.SemaphoreType.DMA,
            pltpu.SemaphoreType.DMA,
            pltpu.VMEM_SHARED((4, REGIONS * P), jnp.float32),
        ],
    )
    return fn(x, va)


def kernel(x, w_sorter, b_sorter, w1, b1, w2, b2, w3, b3, w5, b5):
    val_activa, id_activa = _sorter(x, w_sorter, b_sorter)
    id_activa = id_activa.reshape(B, N)
    cube, spidx, cab = _sc_call(x, val_activa)
    sp_cube = cube.reshape(B, F, REGIONS, P)
    sp_idx = spidx.reshape(B, CH, REGIONS, P)
    cabins = cab.reshape(B, F, REGIONS, NUM_CABIN)
    return (sp_cube, sp_idx, cabins, id_activa)
